# baseline (speedup 1.0000x reference)
import jax
import jax.numpy as jnp
from jax import lax
from jax.experimental import pallas as pl
from jax.experimental.pallas import tpu as pltpu
from jax.experimental.pallas import tpu_sc as plsc

_NC, _NS, _L = 2, 16, 16
_NW = _NC * _NS

_B, _SEQ, _D = 4, 4096, 1024
_ROWS_PER_W = _SEQ // _NW
_CH = 32
_NCH = _ROWS_PER_W // _CH
_UNROLL = 8


def _sc_body(x_hbm, pos_hbm, out_hbm, pos_v, x_v):
    wid = lax.axis_index("s") * _NC + lax.axis_index("c")
    base = wid * _ROWS_PER_W * _D
    n = _CH * _D
    for c in range(_NCH):
        pos_off = base + c * n
        pltpu.sync_copy(pos_hbm.at[pl.ds(pos_off, n)], pos_v)
        for b in range(_B):
            x_off = b * _SEQ * _D + pos_off
            pltpu.sync_copy(x_hbm.at[pl.ds(x_off, n)], x_v)

            def body(i, carry):
                v0 = i * (_L * _UNROLL)
                for u in range(_UNROLL):
                    sl = pl.ds(v0 + u * _L, _L)
                    x_v[sl] = x_v[sl] + pos_v[sl]
                return carry

            lax.fori_loop(0, n // (_L * _UNROLL), body, 0)
            pltpu.sync_copy(x_v, out_hbm.at[pl.ds(x_off, n)])


def kernel(x, pos_table):
    batch, seq_len, d_model = x.shape
    xf = x.reshape(-1)
    pf = pos_table.reshape(-1)
    k = pl.kernel(
        _sc_body,
        out_type=jax.ShapeDtypeStruct((batch * seq_len * d_model,), x.dtype),
        mesh=plsc.VectorSubcoreMesh(core_axis_name="c", subcore_axis_name="s"),
        scratch_types=[
            pltpu.VMEM((_CH * _D,), jnp.float32),
            pltpu.VMEM((_CH * _D,), jnp.float32),
        ],
    )
    return k(xf, pf).reshape(batch, seq_len, d_model)


# speedup vs baseline: 3.7095x; 3.7095x over previous
import jax
import jax.numpy as jnp
from jax.experimental import pallas as pl

_BS = 256


def _add_body(x_ref, pos_ref, o_ref):
    o_ref[...] = x_ref[...] + pos_ref[...][None]


def kernel(x, pos_table):
    batch, seq_len, d_model = x.shape
    grid = (seq_len // _BS, batch)
    return pl.pallas_call(
        _add_body,
        grid=grid,
        in_specs=[
            pl.BlockSpec((1, _BS, d_model), lambda s, b: (b, s, 0)),
            pl.BlockSpec((_BS, d_model), lambda s, b: (s, 0)),
        ],
        out_specs=pl.BlockSpec((1, _BS, d_model), lambda s, b: (b, s, 0)),
        out_shape=jax.ShapeDtypeStruct(x.shape, x.dtype),
    )(x, pos_table)


# speedup vs baseline: 5.4011x; 1.4560x over previous
import jax
import jax.numpy as jnp
from jax.experimental import pallas as pl

_BS = 1024


def _add_body(x_ref, pos_ref, o_ref):
    o_ref[...] = x_ref[...] + pos_ref[...][None]


def kernel(x, pos_table):
    batch, seq_len, d_model = x.shape
    grid = (seq_len // _BS, batch)
    return pl.pallas_call(
        _add_body,
        grid=grid,
        in_specs=[
            pl.BlockSpec((1, _BS, d_model), lambda s, b: (b, s, 0)),
            pl.BlockSpec((_BS, d_model), lambda s, b: (s, 0)),
        ],
        out_specs=pl.BlockSpec((1, _BS, d_model), lambda s, b: (b, s, 0)),
        out_shape=jax.ShapeDtypeStruct(x.shape, x.dtype),
    )(x, pos_table)


# speedup vs baseline: 5.7256x; 1.0601x over previous
import jax
import jax.numpy as jnp
from jax.experimental import pallas as pl

_BS = 2048


def _add_body(x_ref, pos_ref, o_ref):
    o_ref[...] = x_ref[...] + pos_ref[...][None]


def kernel(x, pos_table):
    batch, seq_len, d_model = x.shape
    grid = (seq_len // _BS, batch)
    return pl.pallas_call(
        _add_body,
        grid=grid,
        in_specs=[
            pl.BlockSpec((1, _BS, d_model), lambda s, b: (b, s, 0)),
            pl.BlockSpec((_BS, d_model), lambda s, b: (s, 0)),
        ],
        out_specs=pl.BlockSpec((1, _BS, d_model), lambda s, b: (b, s, 0)),
        out_shape=jax.ShapeDtypeStruct(x.shape, x.dtype),
    )(x, pos_table)
